# baseline (speedup 1.0000x reference)
import jax
import jax.numpy as jnp
from jax import lax
from jax.experimental import pallas as pl
from jax.experimental.pallas import tpu as pltpu
from jax.experimental.pallas import tpu_sc as plsc

N_NODES = 10000
D = 128
N_EDGES = 320000

NUM_TILES = 32
FPT = D // NUM_TILES
EC = 4000
NCHUNK = N_EDGES // EC


def _sc_body(xt_hbm, src_hbm, dst_hbm, acc_hbm,
             xv0, xv1, xv2, xv3, av0, av1, av2, av3, srcv, dstv,
             sem0, sem1):
    cid = lax.axis_index("c")
    sid = lax.axis_index("s")
    wid = sid * 2 + cid
    seg = FPT * N_NODES

    xvs = [xv0, xv1, xv2, xv3]
    avs = [av0, av1, av2, av3]
    sems = [sem0, sem1]

    for f in range(FPT):
        pltpu.sync_copy(
            xt_hbm.at[pl.ds(wid * seg + f * N_NODES, N_NODES)], xvs[f])

    neg_inf16 = jnp.full((16,), -jnp.inf, dtype=jnp.float32)

    def init_acc(r, carry):
        for f in range(FPT):
            avs[f][pl.ds(r * 16, 16)] = neg_inf16
        return carry

    lax.fori_loop(0, N_NODES // 16, init_acc, 0)

    def fire(ch, slot):
        ebase = ch * EC
        sem = sems[slot]
        pltpu.async_copy(src_hbm.at[pl.ds(ebase, EC)],
                         srcv.at[pl.ds(slot * EC, EC)], sem)
        pltpu.async_copy(dst_hbm.at[pl.ds(ebase, EC)],
                         dstv.at[pl.ds(slot * EC, EC)], sem)

    def drain(slot):
        sem = sems[slot]
        pltpu.make_async_copy(src_hbm.at[pl.ds(0, EC)],
                              srcv.at[pl.ds(slot * EC, EC)], sem).wait()
        pltpu.make_async_copy(dst_hbm.at[pl.ds(0, EC)],
                              dstv.at[pl.ds(slot * EC, EC)], sem).wait()

    def process(slot):
        base = slot * EC

        def step(j, resid):
            sv = srcv[pl.ds(base + j * 16, 16)]
            dv = dstv[pl.ds(base + j * 16, 16)]
            vals = [plsc.load_gather(xvs[f], [sv]) for f in range(FPT)]

            for f in range(FPT):
                cur = plsc.load_gather(avs[f], [dv])
                plsc.store_scatter(avs[f], [dv],
                                   jnp.maximum(cur, vals[f]))

            pend = []
            for f in range(FPT):
                back = plsc.load_gather(avs[f], [dv])
                pend.append(back < vals[f])
            for f in range(FPT):
                plsc.store_scatter(avs[f], [dv], vals[f], mask=pend[f])

            cnt, _ = plsc.scan_count(dv)
            return resid | (cnt >= 2).astype(jnp.int32)

        resid = lax.fori_loop(0, EC // 16, step,
                              jnp.zeros((16,), dtype=jnp.int32))
        nres = plsc.all_reduce_population_count(resid != 0)

        @pl.when(nres[0] > 0)
        def _():
            def fixstep(j, carry2):
                sv = srcv[pl.ds(base + j * 16, 16)]
                dv = dstv[pl.ds(base + j * 16, 16)]
                vals = [plsc.load_gather(xvs[f], [sv]) for f in range(FPT)]
                pend = []
                anyp = jnp.zeros((16,), dtype=jnp.bool_)
                for f in range(FPT):
                    back = plsc.load_gather(avs[f], [dv])
                    p = back < vals[f]
                    pend.append(p)
                    anyp = anyp | p
                npend = plsc.all_reduce_population_count(anyp)

                def rcond(c):
                    return c[0] > 0

                def rbody(c):
                    nps = []
                    na = jnp.zeros((16,), dtype=jnp.bool_)
                    for f in range(FPT):
                        psf = c[1 + f] != 0
                        plsc.store_scatter(avs[f], [dv], vals[f], mask=psf)
                        back = plsc.load_gather(avs[f], [dv])
                        p = psf & (back < vals[f])
                        nps.append(p.astype(jnp.int32))
                        na = na | p
                    nn = plsc.all_reduce_population_count(na)
                    return (nn[0],) + tuple(nps)

                lax.while_loop(rcond, rbody,
                               (npend[0],)
                               + tuple(p.astype(jnp.int32) for p in pend))
                return carry2

            lax.fori_loop(0, EC // 16, fixstep, 0)

    fire(0, 0)

    def pair_body(i, carry):
        ch0 = pl.multiple_of(i * 2, 2)
        drain(0)
        fire(ch0 + 1, 1)
        process(0)
        drain(1)

        @pl.when(ch0 + 2 < NCHUNK)
        def _():
            fire(ch0 + 2, 0)

        process(1)
        return carry

    lax.fori_loop(0, NCHUNK // 2, pair_body, 0)

    for f in range(FPT):
        pltpu.sync_copy(
            avs[f], acc_hbm.at[pl.ds(wid * seg + f * N_NODES, N_NODES)])


def _segment_max_sc(xt, src, dst):
    mesh = plsc.VectorSubcoreMesh(core_axis_name="c", subcore_axis_name="s",
                                  num_cores=2, num_subcores=16)
    return pl.kernel(
        _sc_body,
        out_type=jax.ShapeDtypeStruct((NUM_TILES * FPT * N_NODES,), jnp.float32),
        mesh=mesh,
        scratch_types=[
            pltpu.VMEM((N_NODES,), jnp.float32),
            pltpu.VMEM((N_NODES,), jnp.float32),
            pltpu.VMEM((N_NODES,), jnp.float32),
            pltpu.VMEM((N_NODES,), jnp.float32),
            pltpu.VMEM((N_NODES,), jnp.float32),
            pltpu.VMEM((N_NODES,), jnp.float32),
            pltpu.VMEM((N_NODES,), jnp.float32),
            pltpu.VMEM((N_NODES,), jnp.float32),
            pltpu.VMEM((2 * EC,), jnp.int32),
            pltpu.VMEM((2 * EC,), jnp.int32),
            pltpu.SemaphoreType.DMA,
            pltpu.SemaphoreType.DMA,
        ],
        compiler_params=pltpu.CompilerParams(needs_layout_passes=False),
    )(xt, src, dst)


def _dense_body(x_ref, m_ref, w_ref, b_ref, o_ref):
    xb = x_ref[...]
    mb = m_ref[...]
    xj = jnp.where(jnp.isneginf(mb), 0.0, mb - xb)
    h = jnp.dot(xb, w_ref[0:D, :], preferred_element_type=jnp.float32)
    h = h + jnp.dot(xj, w_ref[D:2 * D, :], preferred_element_type=jnp.float32)
    o_ref[...] = jnp.maximum(h + b_ref[...], 0.0)


def _dense_tc(x, m, W, b):
    blk = 400
    grid = N_NODES // blk
    return pl.pallas_call(
        _dense_body,
        out_shape=jax.ShapeDtypeStruct((N_NODES, D), jnp.float32),
        grid=(grid,),
        in_specs=[
            pl.BlockSpec((blk, D), lambda i: (i, 0)),
            pl.BlockSpec((blk, D), lambda i: (i, 0)),
            pl.BlockSpec((2 * D, D), lambda i: (0, 0)),
            pl.BlockSpec((1, D), lambda i: (0, 0)),
        ],
        out_specs=pl.BlockSpec((blk, D), lambda i: (i, 0)),
    )(x, m, W, b)


def kernel(x, edge_index, W, b):
    src = edge_index[0].astype(jnp.int32)
    dst = edge_index[1].astype(jnp.int32)
    xt = x.T.reshape(-1)
    acc = _segment_max_sc(xt, src, dst)
    m = acc.reshape(D, N_NODES).T
    return _dense_tc(x, m, W, b.reshape(1, D))


# speedup vs baseline: 1.7700x; 1.7700x over previous
import jax
import jax.numpy as jnp
from jax import lax
from jax.experimental import pallas as pl
from jax.experimental.pallas import tpu as pltpu
from jax.experimental.pallas import tpu_sc as plsc

N_NODES = 10000
D = 128
N_EDGES = 320000

NUM_TILES = 32
FPT = D // NUM_TILES
EC = 4000
NCHUNK = N_EDGES // EC


def _sc_body(xt_hbm, src_hbm, dst_hbm, acc_hbm,
             xv0, xv1, xv2, xv3, av0, av1, av2, av3, srcv, dstv,
             sem0, sem1):
    cid = lax.axis_index("c")
    sid = lax.axis_index("s")
    wid = sid * 2 + cid
    seg = FPT * N_NODES

    xvs = [xv0, xv1, xv2, xv3]
    avs = [av0, av1, av2, av3]
    sems = [sem0, sem1]

    for f in range(FPT):
        pltpu.sync_copy(
            xt_hbm.at[pl.ds(wid * seg + f * N_NODES, N_NODES)], xvs[f])

    neg_inf16 = jnp.full((16,), -jnp.inf, dtype=jnp.float32)

    def init_acc(r, carry):
        for f in range(FPT):
            avs[f][pl.ds(r * 16, 16)] = neg_inf16
        return carry

    lax.fori_loop(0, N_NODES // 16, init_acc, 0)

    def fire(ch, slot):
        ebase = ch * EC
        sem = sems[slot]
        pltpu.async_copy(src_hbm.at[pl.ds(ebase, EC)],
                         srcv.at[pl.ds(slot * EC, EC)], sem)
        pltpu.async_copy(dst_hbm.at[pl.ds(ebase, EC)],
                         dstv.at[pl.ds(slot * EC, EC)], sem)

    def drain(slot):
        sem = sems[slot]
        pltpu.make_async_copy(src_hbm.at[pl.ds(0, EC)],
                              srcv.at[pl.ds(slot * EC, EC)], sem).wait()
        pltpu.make_async_copy(dst_hbm.at[pl.ds(0, EC)],
                              dstv.at[pl.ds(slot * EC, EC)], sem).wait()

    def process(slot):
        base = slot * EC

        def step(j, resid):
            sv = srcv[pl.ds(base + j * 16, 16)]
            dv = dstv[pl.ds(base + j * 16, 16)]
            vals = [plsc.load_gather(xvs[f], [sv]) for f in range(FPT)]

            for f in range(FPT):
                cur = plsc.load_gather(avs[f], [dv])
                plsc.store_scatter(avs[f], [dv],
                                   jnp.maximum(cur, vals[f]))

            pend = []
            for f in range(FPT):
                back = plsc.load_gather(avs[f], [dv])
                pend.append(back < vals[f])
            for f in range(FPT):
                plsc.store_scatter(avs[f], [dv], vals[f], mask=pend[f])

            cnt, _ = plsc.scan_count(dv)
            return resid | (cnt >= 3).astype(jnp.int32)

        resid = lax.fori_loop(0, EC // 16, step,
                              jnp.zeros((16,), dtype=jnp.int32))
        nres = plsc.all_reduce_population_count(resid != 0)

        @pl.when(nres[0] > 0)
        def _():
            def fixstep(j, carry2):
                sv = srcv[pl.ds(base + j * 16, 16)]
                dv = dstv[pl.ds(base + j * 16, 16)]
                vals = [plsc.load_gather(xvs[f], [sv]) for f in range(FPT)]
                pend = []
                anyp = jnp.zeros((16,), dtype=jnp.bool_)
                for f in range(FPT):
                    back = plsc.load_gather(avs[f], [dv])
                    p = back < vals[f]
                    pend.append(p)
                    anyp = anyp | p
                npend = plsc.all_reduce_population_count(anyp)

                def rcond(c):
                    return c[0] > 0

                def rbody(c):
                    nps = []
                    na = jnp.zeros((16,), dtype=jnp.bool_)
                    for f in range(FPT):
                        psf = c[1 + f] != 0
                        plsc.store_scatter(avs[f], [dv], vals[f], mask=psf)
                        back = plsc.load_gather(avs[f], [dv])
                        p = psf & (back < vals[f])
                        nps.append(p.astype(jnp.int32))
                        na = na | p
                    nn = plsc.all_reduce_population_count(na)
                    return (nn[0],) + tuple(nps)

                lax.while_loop(rcond, rbody,
                               (npend[0],)
                               + tuple(p.astype(jnp.int32) for p in pend))
                return carry2

            lax.fori_loop(0, EC // 16, fixstep, 0)

    fire(0, 0)

    def pair_body(i, carry):
        ch0 = pl.multiple_of(i * 2, 2)
        drain(0)
        fire(ch0 + 1, 1)
        process(0)
        drain(1)

        @pl.when(ch0 + 2 < NCHUNK)
        def _():
            fire(ch0 + 2, 0)

        process(1)
        return carry

    lax.fori_loop(0, NCHUNK // 2, pair_body, 0)

    for f in range(FPT):
        pltpu.sync_copy(
            avs[f], acc_hbm.at[pl.ds(wid * seg + f * N_NODES, N_NODES)])


def _segment_max_sc(xt, src, dst):
    mesh = plsc.VectorSubcoreMesh(core_axis_name="c", subcore_axis_name="s",
                                  num_cores=2, num_subcores=16)
    return pl.kernel(
        _sc_body,
        out_type=jax.ShapeDtypeStruct((NUM_TILES * FPT * N_NODES,), jnp.float32),
        mesh=mesh,
        scratch_types=[
            pltpu.VMEM((N_NODES,), jnp.float32),
            pltpu.VMEM((N_NODES,), jnp.float32),
            pltpu.VMEM((N_NODES,), jnp.float32),
            pltpu.VMEM((N_NODES,), jnp.float32),
            pltpu.VMEM((N_NODES,), jnp.float32),
            pltpu.VMEM((N_NODES,), jnp.float32),
            pltpu.VMEM((N_NODES,), jnp.float32),
            pltpu.VMEM((N_NODES,), jnp.float32),
            pltpu.VMEM((2 * EC,), jnp.int32),
            pltpu.VMEM((2 * EC,), jnp.int32),
            pltpu.SemaphoreType.DMA,
            pltpu.SemaphoreType.DMA,
        ],
        compiler_params=pltpu.CompilerParams(needs_layout_passes=False),
    )(xt, src, dst)


def _dense_body(x_ref, m_ref, w_ref, b_ref, o_ref):
    xb = x_ref[...]
    mb = m_ref[...]
    xj = jnp.where(jnp.isneginf(mb), 0.0, mb - xb)
    h = jnp.dot(xb, w_ref[0:D, :], preferred_element_type=jnp.float32)
    h = h + jnp.dot(xj, w_ref[D:2 * D, :], preferred_element_type=jnp.float32)
    o_ref[...] = jnp.maximum(h + b_ref[...], 0.0)


def _dense_tc(x, m, W, b):
    blk = 400
    grid = N_NODES // blk
    return pl.pallas_call(
        _dense_body,
        out_shape=jax.ShapeDtypeStruct((N_NODES, D), jnp.float32),
        grid=(grid,),
        in_specs=[
            pl.BlockSpec((blk, D), lambda i: (i, 0)),
            pl.BlockSpec((blk, D), lambda i: (i, 0)),
            pl.BlockSpec((2 * D, D), lambda i: (0, 0)),
            pl.BlockSpec((1, D), lambda i: (0, 0)),
        ],
        out_specs=pl.BlockSpec((blk, D), lambda i: (i, 0)),
    )(x, m, W, b)


def kernel(x, edge_index, W, b):
    src = edge_index[0].astype(jnp.int32)
    dst = edge_index[1].astype(jnp.int32)
    xt = x.T.reshape(-1)
    acc = _segment_max_sc(xt, src, dst)
    m = acc.reshape(D, N_NODES).T
    return _dense_tc(x, m, W, b.reshape(1, D))


# speedup vs baseline: 1.9068x; 1.0773x over previous
import jax
import jax.numpy as jnp
import numpy as np
from jax import lax
from jax.experimental import pallas as pl
from jax.experimental.pallas import tpu as pltpu
from jax.experimental.pallas import tpu_sc as plsc

N_NODES = 10000
D = 128
N_EDGES = 320000

NUM_TILES = 32
FPT = D // NUM_TILES
PPT = FPT // 2
EC = 4000
NCHUNK = N_EDGES // EC

NINF_PAIR = int(np.array(0xFF80FF80, dtype=np.uint32).view(np.int32))
ONES_PAIR = int(np.array(0x3F803F80, dtype=np.uint32).view(np.int32))


def _sc_body(xt_hbm, src_hbm, dst_hbm, acc_hbm,
             xp0, xp1, ap0, ap1, srcv, dstv, sem0, sem1):
    cid = lax.axis_index("c")
    sid = lax.axis_index("s")
    wid = sid * 2 + cid
    seg = PPT * N_NODES

    xps = [xp0, xp1]
    aps = [ap0, ap1]
    sems = [sem0, sem1]

    for p in range(PPT):
        pltpu.sync_copy(
            xt_hbm.at[pl.ds(wid * seg + p * N_NODES, N_NODES)], xps[p])

    ninf16 = jnp.full((16,), NINF_PAIR, dtype=jnp.int32)

    def init_acc(r, carry):
        for p in range(PPT):
            aps[p][pl.ds(r * 16, 16)] = ninf16
        return carry

    lax.fori_loop(0, N_NODES // 16, init_acc, 0)

    def fire(ch, slot):
        ebase = ch * EC
        sem = sems[slot]
        pltpu.async_copy(src_hbm.at[pl.ds(ebase, EC)],
                         srcv.at[pl.ds(slot * EC, EC)], sem)
        pltpu.async_copy(dst_hbm.at[pl.ds(ebase, EC)],
                         dstv.at[pl.ds(slot * EC, EC)], sem)

    def drain(slot):
        sem = sems[slot]
        pltpu.make_async_copy(src_hbm.at[pl.ds(0, EC)],
                              srcv.at[pl.ds(slot * EC, EC)], sem).wait()
        pltpu.make_async_copy(dst_hbm.at[pl.ds(0, EC)],
                              dstv.at[pl.ds(slot * EC, EC)], sem).wait()

    def process(slot):
        base = slot * EC

        def step(j, resid):
            sv = srcv[pl.ds(base + j * 16, 16)]
            dv = dstv[pl.ds(base + j * 16, 16)]
            vals = [plsc.bitcast(plsc.load_gather(xps[p], [sv]),
                                 jnp.bfloat16) for p in range(PPT)]

            for p in range(PPT):
                cur = plsc.bitcast(plsc.load_gather(aps[p], [dv]),
                                   jnp.bfloat16)
                plsc.store_scatter(
                    aps[p], [dv],
                    plsc.bitcast(jnp.maximum(cur, vals[p]), jnp.int32))

            for p in range(PPT):
                back = plsc.bitcast(plsc.load_gather(aps[p], [dv]),
                                    jnp.bfloat16)
                okh = jnp.where(back >= vals[p],
                                jnp.bfloat16(1.0), jnp.bfloat16(0.0))
                pend = plsc.bitcast(okh, jnp.int32) != ONES_PAIR
                plsc.store_scatter(
                    aps[p], [dv],
                    plsc.bitcast(jnp.maximum(back, vals[p]), jnp.int32),
                    mask=pend)

            cnt, _ = plsc.scan_count(dv)
            return resid | (cnt >= 3).astype(jnp.int32)

        resid = lax.fori_loop(0, EC // 16, step,
                              jnp.zeros((16,), dtype=jnp.int32))
        nres = plsc.all_reduce_population_count(resid != 0)

        @pl.when(nres[0] > 0)
        def _():
            def fixstep(j, carry2):
                sv = srcv[pl.ds(base + j * 16, 16)]
                dv = dstv[pl.ds(base + j * 16, 16)]
                vals = [plsc.bitcast(plsc.load_gather(xps[p], [sv]),
                                     jnp.bfloat16) for p in range(PPT)]

                def pend_of(p):
                    back = plsc.bitcast(plsc.load_gather(aps[p], [dv]),
                                        jnp.bfloat16)
                    okh = jnp.where(back >= vals[p],
                                    jnp.bfloat16(1.0), jnp.bfloat16(0.0))
                    return plsc.bitcast(okh, jnp.int32) != ONES_PAIR

                pend = []
                anyp = jnp.zeros((16,), dtype=jnp.bool_)
                for p in range(PPT):
                    pp = pend_of(p)
                    pend.append(pp)
                    anyp = anyp | pp
                npend = plsc.all_reduce_population_count(anyp)

                def rcond(c):
                    return c[0] > 0

                def rbody(c):
                    nps = []
                    na = jnp.zeros((16,), dtype=jnp.bool_)
                    for p in range(PPT):
                        psf = c[1 + p] != 0
                        back = plsc.bitcast(plsc.load_gather(aps[p], [dv]),
                                            jnp.bfloat16)
                        plsc.store_scatter(
                            aps[p], [dv],
                            plsc.bitcast(jnp.maximum(back, vals[p]),
                                         jnp.int32),
                            mask=psf)
                        pp = psf & pend_of(p)
                        nps.append(pp.astype(jnp.int32))
                        na = na | pp
                    nn = plsc.all_reduce_population_count(na)
                    return (nn[0],) + tuple(nps)

                lax.while_loop(rcond, rbody,
                               (npend[0],)
                               + tuple(p.astype(jnp.int32) for p in pend))
                return carry2

            lax.fori_loop(0, EC // 16, fixstep, 0)

    fire(0, 0)

    def pair_body(i, carry):
        ch0 = pl.multiple_of(i * 2, 2)
        drain(0)
        fire(ch0 + 1, 1)
        process(0)
        drain(1)

        @pl.when(ch0 + 2 < NCHUNK)
        def _():
            fire(ch0 + 2, 0)

        process(1)
        return carry

    lax.fori_loop(0, NCHUNK // 2, pair_body, 0)

    for p in range(PPT):
        pltpu.sync_copy(
            aps[p], acc_hbm.at[pl.ds(wid * seg + p * N_NODES, N_NODES)])


def _segment_max_sc(xt, src, dst):
    mesh = plsc.VectorSubcoreMesh(core_axis_name="c", subcore_axis_name="s",
                                  num_cores=2, num_subcores=16)
    return pl.kernel(
        _sc_body,
        out_type=jax.ShapeDtypeStruct((NUM_TILES * PPT * N_NODES,),
                                      jnp.int32),
        mesh=mesh,
        scratch_types=[
            pltpu.VMEM((N_NODES,), jnp.int32),
            pltpu.VMEM((N_NODES,), jnp.int32),
            pltpu.VMEM((N_NODES,), jnp.int32),
            pltpu.VMEM((N_NODES,), jnp.int32),
            pltpu.VMEM((2 * EC,), jnp.int32),
            pltpu.VMEM((2 * EC,), jnp.int32),
            pltpu.SemaphoreType.DMA,
            pltpu.SemaphoreType.DMA,
        ],
        compiler_params=pltpu.CompilerParams(needs_layout_passes=False),
    )(xt, src, dst)


def _dense_body(x_ref, m_ref, w_ref, b_ref, o_ref):
    xb = x_ref[...]
    mb = m_ref[...].astype(jnp.float32)
    xj = jnp.where(jnp.isneginf(mb), 0.0, mb - xb)
    h = jnp.dot(xb, w_ref[0:D, :], preferred_element_type=jnp.float32)
    h = h + jnp.dot(xj, w_ref[D:2 * D, :], preferred_element_type=jnp.float32)
    o_ref[...] = jnp.maximum(h + b_ref[...], 0.0)


def _dense_tc(x, m, W, b):
    blk = 400
    grid = N_NODES // blk
    return pl.pallas_call(
        _dense_body,
        out_shape=jax.ShapeDtypeStruct((N_NODES, D), jnp.float32),
        grid=(grid,),
        in_specs=[
            pl.BlockSpec((blk, D), lambda i: (i, 0)),
            pl.BlockSpec((blk, D), lambda i: (i, 0)),
            pl.BlockSpec((2 * D, D), lambda i: (0, 0)),
            pl.BlockSpec((1, D), lambda i: (0, 0)),
        ],
        out_specs=pl.BlockSpec((blk, D), lambda i: (i, 0)),
    )(x, m, W, b)


def kernel(x, edge_index, W, b):
    src = edge_index[0].astype(jnp.int32)
    dst = edge_index[1].astype(jnp.int32)
    xb = x.astype(jnp.bfloat16)
    xp = lax.bitcast_convert_type(xb.reshape(N_NODES, D // 2, 2), jnp.int32)
    xt = xp.T.reshape(-1)
    acc = _segment_max_sc(xt, src, dst)
    mp = acc.reshape(D // 2, N_NODES).T
    m = lax.bitcast_convert_type(mp, jnp.bfloat16).reshape(N_NODES, D)
    return _dense_tc(x, m, W, b.reshape(1, D))
